# Initial kernel scaffold; baseline (speedup 1.0000x reference)
import jax
import jax.numpy as jnp
from jax.experimental import pallas as pl


def kernel(grid_nodes, sphere_nodes, edge_feats, senders, receivers, params):
    raise NotImplementedError("write your pallas kernel here")


# speedup vs baseline: 1.6279x; 1.6279x over previous
import functools

import jax
import jax.numpy as jnp
from jax import lax
from jax.experimental import pallas as pl
from jax.experimental.pallas import tpu as pltpu
from jax.experimental.pallas import tpu_sc as plsc

NSP = 259200
NSPH = 2883
SPH_PAD = 2944
LATENT = 128
NE = 600000
NW = 32
B_PER_W = 18944
NE_PAD = NW * B_PER_W
CHUNK = 128
STEPS = 3

_f32 = jnp.float32


def _ln(x, scale, offset):
    mu = jnp.mean(x, axis=-1, keepdims=True)
    var = jnp.mean((x - mu) ** 2, axis=-1, keepdims=True)
    return (x - mu) * lax.rsqrt(var + 1e-5) * scale + offset


def _dot(a, b):
    return jnp.dot(a, b, preferred_element_type=_f32)


def _mlp2_body(x_ref, w1, b1, s1, o1, w2, b2, s2, o2, out_ref):
    h = _dot(x_ref[...], w1[...]) + b1[...]
    h = _ln(jnp.maximum(h, 0.0), s1[...], o1[...])
    o = _dot(h, w2[...]) + b2[...]
    out_ref[...] = _ln(jnp.maximum(o, 0.0), s2[...], o2[...])


def _mlp2(x, w1, b1, s1, o1, w2, b2, s2, o2, block):
    M, K = x.shape
    N = w2.shape[1]
    params = [w1, b1[None, :], s1[None, :], o1[None, :],
              w2, b2[None, :], s2[None, :], o2[None, :]]
    in_specs = [pl.BlockSpec((block, K), lambda i: (i, 0))]
    in_specs += [pl.BlockSpec(p.shape, lambda i: (0, 0)) for p in params]
    return pl.pallas_call(
        _mlp2_body,
        grid=(pl.cdiv(M, block),),
        in_specs=in_specs,
        out_specs=pl.BlockSpec((block, N), lambda i: (i, 0)),
        out_shape=jax.ShapeDtypeStruct((M, N), _f32),
    )(x, *params)


def _edge_body(e_ref, snd_ref, rcv_ref, wa, wb, wc, b1, s1, o1,
               w2, b2, s2, o2, out_ref):
    h = (_dot(e_ref[...], wa[...]) + _dot(snd_ref[...], wb[...])
         + _dot(rcv_ref[...], wc[...]) + b1[...])
    h = _ln(jnp.maximum(h, 0.0), s1[...], o1[...])
    o = _dot(h, w2[...]) + b2[...]
    out_ref[...] = _ln(jnp.maximum(o, 0.0), s2[...], o2[...])


def _edge_mlp(e, snd, rcv, wa, wb, wc, b1, s1, o1, w2, b2, s2, o2, block):
    params = [wa, wb, wc, b1[None, :], s1[None, :], o1[None, :],
              w2, b2[None, :], s2[None, :], o2[None, :]]
    in_specs = [pl.BlockSpec((block, LATENT), lambda i: (i, 0))] * 3
    in_specs += [pl.BlockSpec(p.shape, lambda i: (0, 0)) for p in params]
    return pl.pallas_call(
        _edge_body,
        grid=(NE_PAD // block,),
        in_specs=in_specs,
        out_specs=pl.BlockSpec((block, LATENT), lambda i: (i, 0)),
        out_shape=jax.ShapeDtypeStruct((NE_PAD, LATENT), _f32),
    )(e, snd, rcv, *params)


def _sphere_body(s_ref, p_ref, wa, wb, b1, s1, o1, w2, b2, s2, o2, out_ref):
    agg = p_ref[0] + p_ref[1]
    h = _dot(s_ref[...], wa[...]) + _dot(agg, wb[...]) + b1[...]
    h = _ln(jnp.maximum(h, 0.0), s1[...], o1[...])
    o = _dot(h, w2[...]) + b2[...]
    out_ref[...] = _ln(jnp.maximum(o, 0.0), s2[...], o2[...])


def _sphere_step(sph, partials, wa, wb, b1, s1, o1, w2, b2, s2, o2):
    return pl.pallas_call(
        _sphere_body,
        out_shape=jax.ShapeDtypeStruct((SPH_PAD, LATENT), _f32),
    )(sph, partials, wa, wb, b1[None, :], s1[None, :], o1[None, :],
      w2, b2[None, :], s2[None, :], o2[None, :])


def _sc_gather(grid_tbl, sph_tbl, snd_idx, rcv_idx):
    mesh = plsc.VectorSubcoreMesh(core_axis_name="c", subcore_axis_name="s")
    oshape = jax.ShapeDtypeStruct((NE_PAD, LATENT), _f32)

    @functools.partial(
        pl.kernel, mesh=mesh,
        out_type=(oshape, oshape),
        scratch_types=[pltpu.VMEM((CHUNK,), jnp.int32),
                       pltpu.VMEM((CHUNK, LATENT), _f32),
                       pltpu.SemaphoreType.DMA],
    )
    def k(grid_hbm, sph_hbm, sidx_hbm, ridx_hbm, snd_out, rcv_out,
          idx_v, rows_v, sem):
        wid = lax.axis_index("s") * 2 + lax.axis_index("c")
        base = wid * B_PER_W

        def run(tbl, idx_hbm, out_hbm):
            def body(i, carry):
                off = base + i * CHUNK
                pltpu.sync_copy(idx_hbm.at[pl.ds(off, CHUNK)], idx_v)
                pltpu.async_copy(tbl.at[idx_v], rows_v, sem).wait()
                pltpu.sync_copy(rows_v, out_hbm.at[pl.ds(off, CHUNK)])
                return carry
            lax.fori_loop(0, B_PER_W // CHUNK, body, 0)

        run(grid_hbm, sidx_hbm, snd_out)
        run(sph_hbm, ridx_hbm, rcv_out)

    return k(grid_tbl, sph_tbl, snd_idx, rcv_idx)


def _sc_scatter(e, rcv_idx, zeros_tbl):
    mesh = plsc.VectorSubcoreMesh(core_axis_name="c", subcore_axis_name="s")

    @functools.partial(
        pl.kernel, mesh=mesh,
        out_type=jax.ShapeDtypeStruct((2, SPH_PAD, LATENT), _f32),
        scratch_types=[pltpu.VMEM((CHUNK,), jnp.int32),
                       pltpu.VMEM((CHUNK, LATENT), _f32),
                       pltpu.VMEM_SHARED((SPH_PAD, LATENT), _f32),
                       pltpu.SemaphoreType.DMA],
    )
    def k(e_hbm, ridx_hbm, zero_hbm, out_hbm, idx_v, rows_v, acc_sh, sem):
        c = lax.axis_index("c")
        s = lax.axis_index("s")
        wid = s * 2 + c
        base = wid * B_PER_W

        @pl.when(s == 0)
        def _():
            pltpu.sync_copy(zero_hbm, acc_sh)
        plsc.subcore_barrier()

        def body(i, carry):
            off = base + i * CHUNK
            pltpu.sync_copy(ridx_hbm.at[pl.ds(off, CHUNK)], idx_v)
            pltpu.sync_copy(e_hbm.at[pl.ds(off, CHUNK)], rows_v)
            pltpu.sync_copy(rows_v, acc_sh.at[idx_v], add=True)
            return carry
        lax.fori_loop(0, B_PER_W // CHUNK, body, 0)
        plsc.subcore_barrier()

        nout = SPH_PAD // CHUNK
        @pl.when(s < nout)
        def _():
            pltpu.sync_copy(acc_sh.at[pl.ds(s * CHUNK, CHUNK)],
                            out_hbm.at[c, pl.ds(s * CHUNK, CHUNK)])

        @pl.when(s + 16 < nout)
        def _():
            pltpu.sync_copy(acc_sh.at[pl.ds((s + 16) * CHUNK, CHUNK)],
                            out_hbm.at[c, pl.ds((s + 16) * CHUNK, CHUNK)])

    return k(e, rcv_idx, zeros_tbl)


def kernel(grid_nodes, sphere_nodes, edge_feats, senders, receivers, params):
    pad_e = NE_PAD - NE
    snd_idx = jnp.concatenate(
        [senders.astype(jnp.int32), jnp.zeros((pad_e,), jnp.int32)])
    rcv_idx = jnp.concatenate(
        [receivers.astype(jnp.int32), jnp.full((pad_e,), NSPH, jnp.int32)])
    ef = jnp.concatenate([edge_feats, jnp.zeros((pad_e, 3), _f32)], axis=0)
    sph_in = jnp.concatenate(
        [sphere_nodes, jnp.zeros((SPH_PAD - NSPH, LATENT), _f32)], axis=0)
    zeros_tbl = jnp.zeros((SPH_PAD, LATENT), _f32)

    def unpack(p):
        return p['W'], p['b'], p['scale'], p['offset']

    ps1, ps2 = params['sender']
    g = _mlp2(grid_nodes, *unpack(ps1), *unpack(ps2), block=1024)
    pr1, pr2 = params['receiver']
    sph = _mlp2(sph_in, *unpack(pr1), *unpack(pr2), block=SPH_PAD)
    pe1, pe2 = params['edge0']
    e = _mlp2(ef, *unpack(pe1), *unpack(pe2), block=1024)

    for t in range(STEPS):
        pedge1, pedge2 = params['edge'][t]
        pnode1, pnode2 = params['node'][t]
        snd_buf, rcv_buf = _sc_gather(g, sph, snd_idx, rcv_idx)
        w1 = pedge1['W']
        e = _edge_mlp(e, snd_buf, rcv_buf,
                      w1[:LATENT], w1[LATENT:2 * LATENT], w1[2 * LATENT:],
                      pedge1['b'], pedge1['scale'], pedge1['offset'],
                      *unpack(pedge2), block=1024)
        partials = _sc_scatter(e, rcv_idx, zeros_tbl)
        wn = pnode1['W']
        g = _mlp2(g, wn[:LATENT], pnode1['b'], pnode1['scale'],
                  pnode1['offset'], *unpack(pnode2), block=1024)
        sph = _sphere_step(sph, partials, wn[:LATENT], wn[LATENT:],
                           pnode1['b'], pnode1['scale'], pnode1['offset'],
                           *unpack(pnode2))

    return jnp.concatenate([g, sph[:NSPH]], axis=0)
